# baseline (speedup 1.0000x reference)
import functools

import jax
import jax.numpy as jnp
from jax import lax
from jax.experimental import pallas as pl
from jax.experimental.pallas import tpu as pltpu
from jax.experimental.pallas import tpu_sc as plsc

_D = 64
_SCALE = 8.0
_NC = 2
_NS = 16
_NW = _NC * _NS
_K = 40
_NBUF = 5
_PF = 2


@functools.lru_cache(maxsize=None)
def _build(batch, seq):
    rows_per_w = batch // _NW
    assert seq == _NBUF * _K
    mesh = plsc.VectorSubcoreMesh(core_axis_name="c", subcore_axis_name="s")

    @functools.partial(
        pl.kernel,
        mesh=mesh,
        out_type=jax.ShapeDtypeStruct((batch, seq, _D), jnp.float32),
        compiler_params=pltpu.CompilerParams(use_tc_tiling_on_sc=False),
        scratch_types=[
            pltpu.VMEM((rows_per_w, seq), jnp.int32),
            pltpu.VMEM((_NBUF, _K, _D), jnp.float32),
            pltpu.SemaphoreType.DMA((_NBUF,)),
            pltpu.SemaphoreType.DMA((_NBUF,)),
        ],
    )
    def gather_scale(x_hbm, table_hbm, out_hbm, idx_v, rows_v, g_sem, s_sem):
        wid = lax.axis_index("s") * _NC + lax.axis_index("c")
        row_base = wid * rows_per_w
        pltpu.sync_copy(x_hbm.at[pl.ds(row_base, rows_per_w)], idx_v)

        def gather_start(bi, c):
            pltpu.async_copy(
                table_hbm.at[idx_v.at[bi, pl.ds(c * _K, _K)]],
                rows_v.at[c],
                g_sem.at[c],
            )

        def gather_wait(c):
            pltpu.make_async_copy(
                table_hbm.at[idx_v.at[0, pl.ds(0, _K)]],
                rows_v.at[c],
                g_sem.at[c],
            ).wait()

        def scatter_start(bi, c):
            pltpu.async_copy(
                rows_v.at[c],
                out_hbm.at[row_base + bi, pl.ds(c * _K, _K)],
                s_sem.at[c],
            )

        def scatter_wait(c):
            pltpu.make_async_copy(
                rows_v.at[c],
                out_hbm.at[0, pl.ds(0, _K)],
                s_sem.at[c],
            ).wait()

        for c in range(_PF):
            gather_start(0, c)

        def per_row(bi, carry):
            for b in range(_NBUF):
                gather_wait(b)

                def scale_rows(r0, c2):
                    for ur in range(4):
                        r = r0 * 4 + ur
                        for u in range(_D // 16):
                            rows_v[b, r, pl.ds(u * 16, 16)] = (
                                rows_v[b, r, pl.ds(u * 16, 16)] * _SCALE
                            )
                    return c2

                lax.fori_loop(0, _K // 4, scale_rows, 0)
                scatter_start(bi, b)

                cp = (b + _PF) % _NBUF
                if b + _PF < _NBUF:
                    @pl.when(bi >= 1)
                    def _():
                        scatter_wait(cp)

                    gather_start(bi, cp)
                else:
                    @pl.when(bi + 1 < rows_per_w)
                    def _():
                        scatter_wait(cp)
                        gather_start(bi + 1, cp)

            return carry

        lax.fori_loop(0, rows_per_w, per_row, 0)

        for b in range(_NBUF):
            scatter_wait(b)

    return gather_scale


def kernel(x, embedding):
    b, s = x.shape
    return _build(b, s)(x.astype(jnp.int32), embedding)
